# Initial kernel scaffold; baseline (speedup 1.0000x reference)
import jax
import jax.numpy as jnp
from jax.experimental import pallas as pl


def kernel(sender_input, gumbel, labels, W_in, b_in, Wx_s, Wh_s, b_s, W_out, b_out, E_s, e_sos, E_r, Wx_r, Wh_r, b_r, W_fc, b_fc):
    raise NotImplementedError("write your pallas kernel here")


# speedup vs baseline: 2.1878x; 2.1878x over previous
import jax
import jax.numpy as jnp
from jax import lax
from jax.experimental import pallas as pl
from jax.experimental.pallas import tpu as pltpu

_ERROR_P = 0.1
_BLK = 256


def _fused_body(x_ref, g_ref, lab_ref, W_in_ref, b_in_ref, Wx_s_ref, Wh_s_ref,
                b_s_ref, W_out_ref, b_out_ref, E_s_ref, e_sos_ref, E_r_ref,
                Wx_r_ref, Wh_r_ref, b_r_ref, W_fc_ref, b_fc_ref, loss_ref):
    blk, n_feat = x_ref.shape
    hidden = W_in_ref.shape[1]
    vocab = W_out_ref.shape[1]
    max_len = g_ref.shape[1]

    x = x_ref[:]
    h_s = jnp.tanh(jnp.dot(x, W_in_ref[:], preferred_element_type=jnp.float32)
                   + b_in_ref[:][None, :])
    e_t = jnp.broadcast_to(e_sos_ref[:][None, :], (blk, hidden))

    E_r = E_r_ref[:]
    er_last = E_r[vocab, :]
    row_ids = lax.broadcasted_iota(jnp.int32, (vocab, hidden), 0)
    M = jnp.where(row_ids == 0,
                  E_r[0:vocab, :] - _ERROR_P * er_last[None, :],
                  (1.0 - _ERROR_P) * E_r[0:vocab, :])
    x_r_off = _ERROR_P * er_last[None, :]

    lab = lab_ref[:]
    feat_ids = lax.broadcasted_iota(jnp.int32, (blk, n_feat), 1)
    onehot = feat_ids == lab

    Wx_s = Wx_s_ref[:]
    Wh_s = Wh_s_ref[:]
    b_s = b_s_ref[:][None, :]
    W_out = W_out_ref[:]
    b_out = b_out_ref[:][None, :]
    E_s = E_s_ref[:]
    Wx_r = Wx_r_ref[:]
    Wh_r = Wh_r_ref[:]
    b_r = b_r_ref[:][None, :]
    W_fc = W_fc_ref[:]
    b_fc = b_fc_ref[:][None, :]

    h_r = jnp.zeros((blk, hidden), dtype=jnp.float32)
    loss = jnp.zeros((blk, 1), dtype=jnp.float32)
    not_eosed = jnp.ones((blk, 1), dtype=jnp.float32)
    nll = jnp.zeros((blk, 1), dtype=jnp.float32)

    for t in range(max_len):
        h_s = jnp.tanh(jnp.dot(e_t, Wx_s, preferred_element_type=jnp.float32)
                       + jnp.dot(h_s, Wh_s, preferred_element_type=jnp.float32)
                       + b_s)
        logits = jnp.dot(h_s, W_out, preferred_element_type=jnp.float32) + b_out
        z = logits + g_ref[:, t, :]
        z = z - jnp.max(z, axis=1, keepdims=True)
        ez = jnp.exp(z)
        sample = ez / jnp.sum(ez, axis=1, keepdims=True)
        eos = sample[:, 0:1]

        x_r = jnp.dot(sample, M, preferred_element_type=jnp.float32) + x_r_off
        h_r = jnp.tanh(jnp.dot(x_r, Wx_r, preferred_element_type=jnp.float32)
                       + jnp.dot(h_r, Wh_r, preferred_element_type=jnp.float32)
                       + b_r)
        out_logits = jnp.dot(h_r, W_fc, preferred_element_type=jnp.float32) + b_fc
        m2 = jnp.max(out_logits, axis=1, keepdims=True)
        lse = jnp.log(jnp.sum(jnp.exp(out_logits - m2), axis=1, keepdims=True)) + m2
        picked = jnp.sum(jnp.where(onehot, out_logits, 0.0), axis=1, keepdims=True)
        nll = lse - picked

        loss = loss + eos * not_eosed * nll
        not_eosed = not_eosed * (1.0 - eos)
        if t + 1 < max_len:
            e_t = jnp.dot(sample, E_s, preferred_element_type=jnp.float32)

    loss = loss + not_eosed * nll
    loss_ref[:] = jnp.broadcast_to(loss, (blk, 128))


def kernel(sender_input, gumbel, labels, W_in, b_in, Wx_s, Wh_s, b_s, W_out,
           b_out, E_s, e_sos, E_r, Wx_r, Wh_r, b_r, W_fc, b_fc):
    B, n_feat = sender_input.shape
    hidden = W_in.shape[1]
    vocab = W_out.shape[1]
    max_len = gumbel.shape[1]
    blk = _BLK

    labels2 = labels.astype(jnp.int32).reshape(B, 1)
    full = lambda shape: pl.BlockSpec(shape, lambda i: (0,) * len(shape))

    out = pl.pallas_call(
        _fused_body,
        grid=(B // blk,),
        in_specs=[
            pl.BlockSpec((blk, n_feat), lambda i: (i, 0)),
            pl.BlockSpec((blk, max_len, vocab), lambda i: (i, 0, 0)),
            pl.BlockSpec((blk, 1), lambda i: (i, 0)),
            full((n_feat, hidden)),
            full((hidden,)),
            full((hidden, hidden)),
            full((hidden, hidden)),
            full((hidden,)),
            full((hidden, vocab)),
            full((vocab,)),
            full((vocab, hidden)),
            full((hidden,)),
            full((vocab + 1, hidden)),
            full((hidden, hidden)),
            full((hidden, hidden)),
            full((hidden,)),
            full((hidden, n_feat)),
            full((n_feat,)),
        ],
        out_specs=pl.BlockSpec((blk, 128), lambda i: (i, 0)),
        out_shape=jax.ShapeDtypeStruct((B, 128), jnp.float32),
        compiler_params=pltpu.CompilerParams(
            dimension_semantics=("parallel",),
        ),
    )(sender_input, gumbel, labels2, W_in, b_in, Wx_s, Wh_s, b_s, W_out,
      b_out, E_s, e_sos, E_r, Wx_r, Wh_r, b_r, W_fc, b_fc)
    return out[:, 0]


# speedup vs baseline: 2.4330x; 1.1121x over previous
import jax
import jax.numpy as jnp
from jax import lax
from jax.experimental import pallas as pl
from jax.experimental.pallas import tpu as pltpu

_ERROR_P = 0.1
_BLK = 256


def _dot(a, b):
    return jnp.dot(a, b, preferred_element_type=jnp.float32)


def _fused_body(x_ref, g_ref, lab_ref, W_in_ref, b_in_ref, Wx_s_ref, Wh_s_ref,
                b_s_ref, W_out_ref, b_out_ref, E_s_ref, e_sos_ref, E_r_ref,
                Wx_r_ref, Wh_r_ref, b_r_ref, W_fc_ref, b_fc_ref, loss_ref):
    blk, n_feat = x_ref.shape
    hidden = W_in_ref.shape[1]
    vocab = W_out_ref.shape[1]
    max_len = g_ref.shape[1]

    E_r = E_r_ref[:]
    er_last = E_r[vocab, :][None, :]
    row_ids = lax.broadcasted_iota(jnp.int32, (vocab, hidden), 0)
    M = jnp.where(row_ids == 0,
                  E_r[0:vocab, :] - _ERROR_P * er_last,
                  (1.0 - _ERROR_P) * E_r[0:vocab, :])

    Wx_s = Wx_s_ref[:]
    Wx_r = Wx_r_ref[:]
    Wc1 = jnp.concatenate([Wh_s_ref[:], W_out_ref[:]], axis=1)
    Wc2 = jnp.concatenate([_dot(E_s_ref[:], Wx_s), _dot(M, Wx_r)], axis=1)
    Wc3 = jnp.concatenate([Wh_r_ref[:], W_fc_ref[:]], axis=1)

    b_s = b_s_ref[:][None, :]
    b_out = b_out_ref[:][None, :]
    b_r2 = b_r_ref[:][None, :] + _dot(_ERROR_P * er_last, Wx_r)
    b_fc = b_fc_ref[:][None, :]

    lab = lab_ref[:]
    feat_ids = lax.broadcasted_iota(jnp.int32, (blk, n_feat), 1)
    onehot = feat_ids == lab

    h_s = jnp.tanh(_dot(x_ref[:], W_in_ref[:]) + b_in_ref[:][None, :])
    y = _dot(h_s, Wc1)
    e_part = jnp.broadcast_to(_dot(e_sos_ref[:][None, :], Wx_s), (blk, hidden))
    hr_rec = jnp.zeros((blk, hidden), dtype=jnp.float32)
    loss = jnp.zeros((blk, 1), dtype=jnp.float32)
    not_eosed = jnp.ones((blk, 1), dtype=jnp.float32)
    nll = jnp.zeros((blk, 1), dtype=jnp.float32)

    for t in range(max_len):
        h_s = jnp.tanh(e_part + y[:, :hidden] + b_s)
        y = _dot(h_s, Wc1)
        z = y[:, hidden:] + b_out + g_ref[:, t, :]
        z = z - jnp.max(z, axis=1, keepdims=True)
        ez = jnp.exp(z)
        sample = ez / jnp.sum(ez, axis=1, keepdims=True)
        eos = sample[:, 0:1]

        c = _dot(sample, Wc2)
        e_part = c[:, :hidden]
        h_r = jnp.tanh(c[:, hidden:] + hr_rec + b_r2)
        w = _dot(h_r, Wc3)
        hr_rec = w[:, :hidden]
        out_logits = w[:, hidden:] + b_fc
        m2 = jnp.max(out_logits, axis=1, keepdims=True)
        lse = jnp.log(jnp.sum(jnp.exp(out_logits - m2), axis=1, keepdims=True)) + m2
        picked = jnp.sum(jnp.where(onehot, out_logits, 0.0), axis=1, keepdims=True)
        nll = lse - picked

        loss = loss + eos * not_eosed * nll
        not_eosed = not_eosed * (1.0 - eos)

    loss = loss + not_eosed * nll
    loss_ref[:] = jnp.broadcast_to(loss, (blk, 128))


def kernel(sender_input, gumbel, labels, W_in, b_in, Wx_s, Wh_s, b_s, W_out,
           b_out, E_s, e_sos, E_r, Wx_r, Wh_r, b_r, W_fc, b_fc):
    B, n_feat = sender_input.shape
    hidden = W_in.shape[1]
    vocab = W_out.shape[1]
    max_len = gumbel.shape[1]
    blk = _BLK

    labels2 = labels.astype(jnp.int32).reshape(B, 1)
    full = lambda shape: pl.BlockSpec(shape, lambda i: (0,) * len(shape))

    out = pl.pallas_call(
        _fused_body,
        grid=(B // blk,),
        in_specs=[
            pl.BlockSpec((blk, n_feat), lambda i: (i, 0)),
            pl.BlockSpec((blk, max_len, vocab), lambda i: (i, 0, 0)),
            pl.BlockSpec((blk, 1), lambda i: (i, 0)),
            full((n_feat, hidden)),
            full((hidden,)),
            full((hidden, hidden)),
            full((hidden, hidden)),
            full((hidden,)),
            full((hidden, vocab)),
            full((vocab,)),
            full((vocab, hidden)),
            full((hidden,)),
            full((vocab + 1, hidden)),
            full((hidden, hidden)),
            full((hidden, hidden)),
            full((hidden,)),
            full((hidden, n_feat)),
            full((n_feat,)),
        ],
        out_specs=pl.BlockSpec((blk, 128), lambda i: (i, 0)),
        out_shape=jax.ShapeDtypeStruct((B, 128), jnp.float32),
        compiler_params=pltpu.CompilerParams(
            dimension_semantics=("parallel",),
        ),
    )(sender_input, gumbel, labels2, W_in, b_in, Wx_s, Wh_s, b_s, W_out,
      b_out, E_s, e_sos, E_r, Wx_r, Wh_r, b_r, W_fc, b_fc)
    return out[:, 0]
